# Initial kernel scaffold; baseline (speedup 1.0000x reference)
import jax
import jax.numpy as jnp
from jax.experimental import pallas as pl


def kernel(x, edge_label_index, W, b):
    raise NotImplementedError("write your pallas kernel here")


# speedup vs baseline: 34.5317x; 34.5317x over previous
import functools

import jax
import jax.numpy as jnp
from jax import lax
from jax.experimental import pallas as pl
from jax.experimental.pallas import tpu as pltpu
from jax.experimental.pallas import tpu_sc as plsc

_N_NODES = 10000
_D = 128
_N_EDGES = 320000

_NC = 2
_NS = 16
_NW = _NC * _NS
_LANES = 16
_E_PER = _N_EDGES // _NW
_ITERS = _E_PER // _LANES


def _tc_body(x_ref, wt_ref, bias_ref, out_ref):
    out_ref[...] = (
        jnp.dot(x_ref[...], wt_ref[...], preferred_element_type=jnp.float32)
        + bias_ref[...]
    )


def _node_table(x, W, b):
    wt = W.reshape(2, _D).T
    bias = jnp.stack([b, jnp.zeros_like(b)], axis=1)
    return pl.pallas_call(
        _tc_body,
        out_shape=jax.ShapeDtypeStruct((_N_NODES, 2), jnp.float32),
    )(x, wt, bias)


def _sc_body(table_hbm, edges_hbm, out_hbm, tab_v, src_v, dst_v, out_v):
    wid = lax.axis_index("s") * _NC + lax.axis_index("c")
    base = wid * _E_PER
    pltpu.sync_copy(table_hbm, tab_v)
    pltpu.sync_copy(edges_hbm.at[pl.ds(base, _E_PER)], src_v)
    pltpu.sync_copy(edges_hbm.at[pl.ds(_N_EDGES + base, _E_PER)], dst_v)

    def step(i, carry):
        sl = pl.ds(i * _LANES, _LANES)
        si = src_v[sl]
        di = dst_v[sl]
        pv = plsc.load_gather(tab_v, [si * 2])
        qv = plsc.load_gather(tab_v, [di * 2 + 1])
        out_v[sl] = pv + qv
        return carry

    lax.fori_loop(0, _ITERS, step, 0)
    pltpu.sync_copy(out_v, out_hbm.at[pl.ds(base, _E_PER)])


@functools.cache
def _sc_gather():
    return pl.kernel(
        _sc_body,
        mesh=plsc.VectorSubcoreMesh(core_axis_name="c", subcore_axis_name="s"),
        compiler_params=pltpu.CompilerParams(needs_layout_passes=False),
        out_type=jax.ShapeDtypeStruct((_N_EDGES,), jnp.float32),
        scratch_types=[
            pltpu.VMEM((2 * _N_NODES,), jnp.float32),
            pltpu.VMEM((_E_PER,), jnp.int32),
            pltpu.VMEM((_E_PER,), jnp.int32),
            pltpu.VMEM((_E_PER,), jnp.float32),
        ],
    )


def kernel(x, edge_label_index, W, b):
    table = _node_table(x, W, b).reshape(2 * _N_NODES)
    edges = edge_label_index.astype(jnp.int32).reshape(2 * _N_EDGES)
    out = _sc_gather()(table, edges)
    return out.reshape(_N_EDGES, 1)


# speedup vs baseline: 36.8312x; 1.0666x over previous
import functools

import jax
import jax.numpy as jnp
from jax import lax
from jax.experimental import pallas as pl
from jax.experimental.pallas import tpu as pltpu
from jax.experimental.pallas import tpu_sc as plsc

_N_NODES = 10000
_D = 128
_N_EDGES = 320000

_NC = 2
_NS = 16
_NW = _NC * _NS
_LANES = 16
_E_PER = _N_EDGES // _NW
_ITERS = _E_PER // _LANES


def _tc_body(x_ref, wt_ref, bias_ref, out_ref):
    out_ref[...] = (
        jnp.dot(x_ref[...], wt_ref[...], preferred_element_type=jnp.float32)
        + bias_ref[...]
    )


def _node_table(x, W, b):
    wt = W.reshape(2, _D).T
    bias = jnp.stack([b, jnp.zeros_like(b)], axis=1)
    return pl.pallas_call(
        _tc_body,
        out_shape=jax.ShapeDtypeStruct((_N_NODES, 2), jnp.float32),
    )(x, wt, bias)


def _sc_body(table_hbm, edges_hbm, out_hbm, tab_v, src_v, dst_v, out_v, sem):
    wid = lax.axis_index("s") * _NC + lax.axis_index("c")
    base = wid * _E_PER
    c0 = pltpu.async_copy(table_hbm, tab_v, sem)
    c1 = pltpu.async_copy(edges_hbm.at[pl.ds(base, _E_PER)], src_v, sem)
    c2 = pltpu.async_copy(edges_hbm.at[pl.ds(_N_EDGES + base, _E_PER)], dst_v, sem)
    c0.wait()
    c1.wait()
    c2.wait()

    @plsc.parallel_loop(0, _E_PER, step=_LANES, unroll=8)
    def _(i):
        sl = pl.ds(i, _LANES)
        si = src_v[sl]
        di = dst_v[sl]
        pv = plsc.load_gather(tab_v, [si * 2])
        qv = plsc.load_gather(tab_v, [di * 2 + 1])
        out_v[sl] = pv + qv

    pltpu.sync_copy(out_v, out_hbm.at[pl.ds(base, _E_PER)])


@functools.cache
def _sc_gather():
    return pl.kernel(
        _sc_body,
        mesh=plsc.VectorSubcoreMesh(core_axis_name="c", subcore_axis_name="s"),
        compiler_params=pltpu.CompilerParams(needs_layout_passes=False),
        out_type=jax.ShapeDtypeStruct((_N_EDGES,), jnp.float32),
        scratch_types=[
            pltpu.VMEM((2 * _N_NODES,), jnp.float32),
            pltpu.VMEM((_E_PER,), jnp.int32),
            pltpu.VMEM((_E_PER,), jnp.int32),
            pltpu.VMEM((_E_PER,), jnp.float32),
            pltpu.SemaphoreType.DMA,
        ],
    )


def kernel(x, edge_label_index, W, b):
    table = _node_table(x, W, b).reshape(2 * _N_NODES)
    edges = edge_label_index.astype(jnp.int32).reshape(2 * _N_EDGES)
    out = _sc_gather()(table, edges)
    return out.reshape(_N_EDGES, 1)


# speedup vs baseline: 45.2740x; 1.2292x over previous
import functools

import jax
import jax.numpy as jnp
from jax import lax
from jax.experimental import pallas as pl
from jax.experimental.pallas import tpu as pltpu
from jax.experimental.pallas import tpu_sc as plsc

_N_NODES = 10000
_D = 128
_N_EDGES = 320000

_NC = 2
_NS = 16
_NW = _NC * _NS
_LANES = 16
_E_PER = _N_EDGES // _NW


def _tc_body(x_ref, wt_ref, bias_ref, edges_ref, pq_ref, flat_ref):
    pq_ref[...] = (
        lax.dot_general(
            wt_ref[...], x_ref[...],
            dimension_numbers=(((1,), (1,)), ((), ())),
            preferred_element_type=jnp.float32,
        )
        + bias_ref[...]
    )
    flat_ref[pl.ds(0, _N_EDGES)] = edges_ref[0, :].reshape(_N_EDGES)
    flat_ref[pl.ds(_N_EDGES, _N_EDGES)] = edges_ref[1, :].reshape(_N_EDGES)


def _tc_prep(x, W, b, edges):
    wt = W.reshape(2, _D)
    bias = jnp.stack([b, jnp.zeros_like(b)])
    return pl.pallas_call(
        _tc_body,
        out_shape=(
            jax.ShapeDtypeStruct((2, _N_NODES), jnp.float32),
            jax.ShapeDtypeStruct((2 * _N_EDGES,), jnp.int32),
        ),
    )(x, wt, bias, edges)


def _sc_body(table_hbm, edges_hbm, out_hbm, tab_v, src_v, dst_v, out_v, sem):
    wid = lax.axis_index("s") * _NC + lax.axis_index("c")
    base = wid * _E_PER
    c0 = pltpu.async_copy(table_hbm, tab_v, sem)
    c1 = pltpu.async_copy(edges_hbm.at[pl.ds(base, _E_PER)], src_v, sem)
    c2 = pltpu.async_copy(edges_hbm.at[pl.ds(_N_EDGES + base, _E_PER)], dst_v, sem)
    c0.wait()
    c1.wait()
    c2.wait()

    @plsc.parallel_loop(0, _E_PER, step=_LANES, unroll=8)
    def _(i):
        sl = pl.ds(i, _LANES)
        si = src_v[sl]
        di = dst_v[sl]
        pv = plsc.load_gather(tab_v, [si])
        qv = plsc.load_gather(tab_v, [di + _N_NODES])
        out_v[sl] = pv + qv

    pltpu.sync_copy(out_v, out_hbm.at[pl.ds(base, _E_PER)])


@functools.cache
def _sc_gather():
    return pl.kernel(
        _sc_body,
        mesh=plsc.VectorSubcoreMesh(core_axis_name="c", subcore_axis_name="s"),
        compiler_params=pltpu.CompilerParams(needs_layout_passes=False),
        out_type=jax.ShapeDtypeStruct((_N_EDGES,), jnp.float32),
        scratch_types=[
            pltpu.VMEM((2 * _N_NODES,), jnp.float32),
            pltpu.VMEM((_E_PER,), jnp.int32),
            pltpu.VMEM((_E_PER,), jnp.int32),
            pltpu.VMEM((_E_PER,), jnp.float32),
            pltpu.SemaphoreType.DMA,
        ],
    )


def kernel(x, edge_label_index, W, b):
    pq, edges = _tc_prep(x, W, b, edge_label_index.astype(jnp.int32))
    table = pq.reshape(2 * _N_NODES)
    out = _sc_gather()(table, edges)
    return out.reshape(_N_EDGES, 1)


# speedup vs baseline: 47.2496x; 1.0436x over previous
import functools

import jax
import jax.numpy as jnp
from jax import lax
from jax.experimental import pallas as pl
from jax.experimental.pallas import tpu as pltpu
from jax.experimental.pallas import tpu_sc as plsc

_N_NODES = 10000
_D = 128
_N_EDGES = 320000

_NC = 2
_NS = 16
_NW = _NC * _NS
_LANES = 16
_E_PER = _N_EDGES // _NW


def _tc_body(x_ref, wt_ref, bias_ref, edges_ref, pq_ref, flat_ref):
    bias = jnp.concatenate(
        [bias_ref[...], jnp.zeros_like(bias_ref[...])], axis=0
    )
    pq_ref[...] = (
        lax.dot_general(
            wt_ref[...], x_ref[...],
            dimension_numbers=(((1,), (1,)), ((), ())),
            preferred_element_type=jnp.float32,
        )
        + bias
    )
    packed = edges_ref[0, :] | (edges_ref[1, :] << 16)
    flat_ref[...] = packed.reshape(_N_EDGES)


def _tc_prep(x, W, b, edges):
    wt = W.reshape(2, _D)
    bias = b.reshape(1, 1)
    return pl.pallas_call(
        _tc_body,
        out_shape=(
            jax.ShapeDtypeStruct((2, _N_NODES), jnp.float32),
            jax.ShapeDtypeStruct((_N_EDGES,), jnp.int32),
        ),
    )(x, wt, bias, edges)


def _sc_body(table_hbm, edges_hbm, out_hbm, tab_v, edge_v, out_v, sem):
    wid = lax.axis_index("s") * _NC + lax.axis_index("c")
    base = wid * _E_PER
    c0 = pltpu.async_copy(table_hbm, tab_v, sem)
    c1 = pltpu.async_copy(edges_hbm.at[pl.ds(base, _E_PER)], edge_v, sem)
    c0.wait()
    c1.wait()

    @plsc.parallel_loop(0, _E_PER, step=_LANES, unroll=8)
    def _(i):
        sl = pl.ds(i, _LANES)
        ev = edge_v[sl]
        si = ev & 0xFFFF
        di = lax.shift_right_logical(ev, 16)
        pv = plsc.load_gather(tab_v, [si])
        qv = plsc.load_gather(tab_v, [di + _N_NODES])
        out_v[sl] = pv + qv

    pltpu.sync_copy(out_v, out_hbm.at[pl.ds(base, _E_PER)])


@functools.cache
def _sc_gather():
    return pl.kernel(
        _sc_body,
        mesh=plsc.VectorSubcoreMesh(core_axis_name="c", subcore_axis_name="s"),
        compiler_params=pltpu.CompilerParams(needs_layout_passes=False),
        out_type=jax.ShapeDtypeStruct((_N_EDGES,), jnp.float32),
        scratch_types=[
            pltpu.VMEM((2 * _N_NODES,), jnp.float32),
            pltpu.VMEM((_E_PER,), jnp.int32),
            pltpu.VMEM((_E_PER,), jnp.float32),
            pltpu.SemaphoreType.DMA,
        ],
    )


def kernel(x, edge_label_index, W, b):
    pq, edges = _tc_prep(x, W, b, edge_label_index.astype(jnp.int32))
    table = pq.reshape(2 * _N_NODES)
    out = _sc_gather()(table, edges)
    return out.reshape(_N_EDGES, 1)


# speedup vs baseline: 47.4173x; 1.0035x over previous
import functools

import jax
import jax.numpy as jnp
from jax import lax
from jax.experimental import pallas as pl
from jax.experimental.pallas import tpu as pltpu
from jax.experimental.pallas import tpu_sc as plsc

_N_NODES = 10000
_D = 128
_N_EDGES = 320000

_NC = 2
_NS = 16
_NW = _NC * _NS
_LANES = 16
_E_PER = _N_EDGES // _NW


def _tc_body(x_ref, wt_ref, bias_ref, edges_ref, pq_ref, flat_ref):
    bias = jnp.concatenate(
        [bias_ref[...], jnp.zeros_like(bias_ref[...])], axis=0
    )
    pq_ref[...] = (
        lax.dot_general(
            wt_ref[...], x_ref[...],
            dimension_numbers=(((1,), (1,)), ((), ())),
            preferred_element_type=jnp.float32,
        )
        + bias
    )
    packed = edges_ref[0, :] | (edges_ref[1, :] << 16)
    flat_ref[...] = packed.reshape(_N_EDGES)


def _tc_prep(x, W, b, edges):
    wt = W.reshape(2, _D)
    bias = b.reshape(1, 1)
    return pl.pallas_call(
        _tc_body,
        out_shape=(
            jax.ShapeDtypeStruct((2, _N_NODES), jnp.float32),
            jax.ShapeDtypeStruct((_N_EDGES,), jnp.int32),
        ),
    )(x, wt, bias, edges)


def _sc_body(table_hbm, edges_hbm, out_hbm, tab_v, edge_v, out_v, sem):
    wid = lax.axis_index("s") * _NC + lax.axis_index("c")
    base = wid * _E_PER
    c0 = pltpu.async_copy(table_hbm, tab_v, sem)
    c1 = pltpu.async_copy(edges_hbm.at[pl.ds(base, _E_PER)], edge_v, sem)
    c0.wait()
    c1.wait()

    half = _E_PER // 2

    def run(lo, hi):
        @plsc.parallel_loop(lo, hi, step=_LANES, unroll=8)
        def _(i):
            sl = pl.ds(i, _LANES)
            ev = edge_v[sl]
            si = ev & 0xFFFF
            di = lax.shift_right_logical(ev, 16)
            pv = plsc.load_gather(tab_v, [si])
            qv = plsc.load_gather(tab_v, [di + _N_NODES])
            out_v[sl] = pv + qv

    run(0, half)
    c3 = pltpu.async_copy(
        out_v.at[pl.ds(0, half)], out_hbm.at[pl.ds(base, half)], sem
    )
    run(half, _E_PER)
    c4 = pltpu.async_copy(
        out_v.at[pl.ds(half, half)], out_hbm.at[pl.ds(base + half, half)], sem
    )
    c3.wait()
    c4.wait()


@functools.cache
def _sc_gather():
    return pl.kernel(
        _sc_body,
        mesh=plsc.VectorSubcoreMesh(core_axis_name="c", subcore_axis_name="s"),
        compiler_params=pltpu.CompilerParams(needs_layout_passes=False),
        out_type=jax.ShapeDtypeStruct((_N_EDGES,), jnp.float32),
        scratch_types=[
            pltpu.VMEM((2 * _N_NODES,), jnp.float32),
            pltpu.VMEM((_E_PER,), jnp.int32),
            pltpu.VMEM((_E_PER,), jnp.float32),
            pltpu.SemaphoreType.DMA,
        ],
    )


def kernel(x, edge_label_index, W, b):
    pq, edges = _tc_prep(x, W, b, edge_label_index.astype(jnp.int32))
    table = pq.reshape(2 * _N_NODES)
    out = _sc_gather()(table, edges)
    return out.reshape(_N_EDGES, 1)


# speedup vs baseline: 47.7982x; 1.0080x over previous
import functools

import jax
import jax.numpy as jnp
from jax import lax
from jax.experimental import pallas as pl
from jax.experimental.pallas import tpu as pltpu
from jax.experimental.pallas import tpu_sc as plsc

_N_NODES = 10000
_D = 128
_N_EDGES = 320000

_NC = 2
_NS = 16
_NW = _NC * _NS
_LANES = 16
_E_PER = 10112
_LAST_LO = _N_EDGES - _E_PER


def _tc_body(x_ref, wt_ref, bias_ref, pq_ref):
    bias = jnp.concatenate(
        [bias_ref[...], jnp.zeros_like(bias_ref[...])], axis=0
    )
    pq_ref[...] = (
        lax.dot_general(
            wt_ref[...], x_ref[...],
            dimension_numbers=(((1,), (1,)), ((), ())),
            preferred_element_type=jnp.float32,
        )
        + bias
    )


def _tc_prep(x, W, b):
    wt = W.reshape(2, _D)
    bias = b.reshape(1, 1)
    return pl.pallas_call(
        _tc_body,
        out_shape=jax.ShapeDtypeStruct((2, _N_NODES), jnp.float32),
    )(x, wt, bias)


def _sc_body(table_hbm, edges_hbm, out_hbm, tab_v, win_v, out_v, sem):
    wid = lax.axis_index("s") * _NC + lax.axis_index("c")
    base = jnp.minimum(wid * _E_PER, _LAST_LO)
    c0 = pltpu.async_copy(table_hbm, tab_v, sem)
    c1 = pltpu.async_copy(edges_hbm.at[:, pl.ds(base, _E_PER)], win_v, sem)
    c0.wait()
    c1.wait()

    half = _E_PER // 2

    def run(lo, hi):
        @plsc.parallel_loop(lo, hi, step=_LANES, unroll=8)
        def _(i):
            sl = pl.ds(i, _LANES)
            si = win_v[0, sl]
            di = win_v[1, sl]
            pv = plsc.load_gather(tab_v, [si])
            qv = plsc.load_gather(tab_v, [di + _N_NODES])
            out_v[sl] = pv + qv

    run(0, half)
    c3 = pltpu.async_copy(
        out_v.at[pl.ds(0, half)], out_hbm.at[pl.ds(base, half)], sem
    )
    run(half, _E_PER)
    c4 = pltpu.async_copy(
        out_v.at[pl.ds(half, half)], out_hbm.at[pl.ds(base + half, half)], sem
    )
    c3.wait()
    c4.wait()


@functools.cache
def _sc_gather():
    return pl.kernel(
        _sc_body,
        mesh=plsc.VectorSubcoreMesh(core_axis_name="c", subcore_axis_name="s"),
        compiler_params=pltpu.CompilerParams(needs_layout_passes=False),
        out_type=jax.ShapeDtypeStruct((_N_EDGES,), jnp.float32),
        scratch_types=[
            pltpu.VMEM((2 * _N_NODES,), jnp.float32),
            pltpu.VMEM((2, _E_PER), jnp.int32),
            pltpu.VMEM((_E_PER,), jnp.float32),
            pltpu.SemaphoreType.DMA,
        ],
    )


def kernel(x, edge_label_index, W, b):
    pq = _tc_prep(x, W, b)
    table = pq.reshape(2 * _N_NODES)
    out = _sc_gather()(table, edge_label_index.astype(jnp.int32))
    return out.reshape(_N_EDGES, 1)


# speedup vs baseline: 50.1011x; 1.0482x over previous
import functools

import jax
import jax.numpy as jnp
from jax import lax
from jax.experimental import pallas as pl
from jax.experimental.pallas import tpu as pltpu
from jax.experimental.pallas import tpu_sc as plsc

_N_NODES = 10000
_D = 128
_N_EDGES = 320000

_NC = 2
_NS = 16
_NW = _NC * _NS
_LANES = 16
_E_PER = 10112
_LAST_LO = _N_EDGES - _E_PER
_Q_OFF = 10240


def _tc_body(x_ref, wt_ref, bias_ref, pq_ref):
    bias = jnp.concatenate(
        [bias_ref[...], jnp.zeros_like(bias_ref[...])], axis=0
    )
    pq = (
        lax.dot_general(
            wt_ref[...], x_ref[...],
            dimension_numbers=(((1,), (1,)), ((), ())),
            preferred_element_type=jnp.float32,
        )
        + bias
    )
    pq_ref[pl.ds(0, _N_NODES)] = pq[0:1, :].reshape(_N_NODES)
    pq_ref[pl.ds(_Q_OFF, _N_NODES)] = pq[1:2, :].reshape(_N_NODES)


def _tc_prep(x, W, b):
    wt = W.reshape(2, _D)
    bias = b.reshape(1, 1)
    return pl.pallas_call(
        _tc_body,
        out_shape=jax.ShapeDtypeStruct((_Q_OFF + _N_NODES,), jnp.float32),
    )(x, wt, bias)


def _sc_body(table_hbm, edges_hbm, out_hbm, tab_v, win_v, out_v, sem):
    wid = lax.axis_index("s") * _NC + lax.axis_index("c")
    base = jnp.minimum(wid * _E_PER, _LAST_LO)
    c0 = pltpu.async_copy(table_hbm, tab_v, sem)
    c1 = pltpu.async_copy(edges_hbm.at[:, pl.ds(base, _E_PER)], win_v, sem)
    c0.wait()
    c1.wait()

    half = _E_PER // 2

    def run(lo, hi):
        @plsc.parallel_loop(lo, hi, step=_LANES, unroll=8)
        def _(i):
            sl = pl.ds(i, _LANES)
            si = win_v[0, sl]
            di = win_v[1, sl]
            pv = plsc.load_gather(tab_v, [si])
            qv = plsc.load_gather(tab_v, [di + _Q_OFF])
            out_v[sl] = pv + qv

    run(0, half)
    c3 = pltpu.async_copy(
        out_v.at[pl.ds(0, half)], out_hbm.at[pl.ds(base, half)], sem
    )
    run(half, _E_PER)
    c4 = pltpu.async_copy(
        out_v.at[pl.ds(half, half)], out_hbm.at[pl.ds(base + half, half)], sem
    )
    c3.wait()
    c4.wait()


@functools.cache
def _sc_gather():
    return pl.kernel(
        _sc_body,
        mesh=plsc.VectorSubcoreMesh(core_axis_name="c", subcore_axis_name="s"),
        compiler_params=pltpu.CompilerParams(needs_layout_passes=False),
        out_type=jax.ShapeDtypeStruct((_N_EDGES,), jnp.float32),
        scratch_types=[
            pltpu.VMEM((_Q_OFF + _N_NODES,), jnp.float32),
            pltpu.VMEM((2, _E_PER), jnp.int32),
            pltpu.VMEM((_E_PER,), jnp.float32),
            pltpu.SemaphoreType.DMA,
        ],
    )


def kernel(x, edge_label_index, W, b):
    table = _tc_prep(x, W, b)
    out = _sc_gather()(table, edge_label_index.astype(jnp.int32))
    return out.reshape(_N_EDGES, 1)


# speedup vs baseline: 63.9475x; 1.2764x over previous
import functools

import jax
import jax.numpy as jnp
from jax import lax
from jax.experimental import pallas as pl
from jax.experimental.pallas import tpu as pltpu
from jax.experimental.pallas import tpu_sc as plsc

_N_NODES = 10000
_D = 128
_N_EDGES = 320000

_NC = 2
_NS = 16
_NW = _NC * _NS
_LANES = 16
_E_PER = 10112
_LAST_LO = _N_EDGES - _E_PER
_Q_OFF = 10240


def _tc_body(x_ref, wt_ref, bias_ref, pq_ref):
    bias = jnp.concatenate(
        [bias_ref[...], jnp.zeros_like(bias_ref[...])], axis=0
    )
    pq = (
        lax.dot_general(
            wt_ref[...], x_ref[...],
            dimension_numbers=(((1,), (1,)), ((), ())),
            preferred_element_type=jnp.float32,
        )
        + bias
    )
    pq_ref[pl.ds(0, _N_NODES)] = pq[0:1, :].reshape(_N_NODES)
    pq_ref[pl.ds(_Q_OFF, _N_NODES)] = pq[1:2, :].reshape(_N_NODES)


def _tc_prep(x, W, b):
    wt = W.reshape(2, _D)
    bias = b.reshape(1, 1)
    return pl.pallas_call(
        _tc_body,
        out_shape=jax.ShapeDtypeStruct((_Q_OFF + _N_NODES,), jnp.float32),
    )(x, wt, bias)


def _sc_body(table_hbm, edges_hbm, out_hbm, tab_v, win_v, out_v, sem):
    wid = lax.axis_index("s") * _NC + lax.axis_index("c")
    base = jnp.minimum(wid * _E_PER, _LAST_LO)
    c0 = pltpu.async_copy(table_hbm, tab_v, sem)
    c1 = pltpu.async_copy(edges_hbm.at[:, pl.ds(base, _E_PER)], win_v, sem)
    c0.wait()
    c1.wait()

    @plsc.parallel_loop(0, _E_PER, step=_LANES, unroll=8)
    def _(i):
        sl = pl.ds(i, _LANES)
        si = win_v[0, sl]
        di = win_v[1, sl]
        pv = plsc.load_gather(tab_v, [si])
        qv = plsc.load_gather(tab_v, [di + _Q_OFF])
        out_v[sl] = pv + qv

    pltpu.sync_copy(out_v, out_hbm.at[0, pl.ds(base, _E_PER)])


@functools.cache
def _sc_gather():
    return pl.kernel(
        _sc_body,
        mesh=plsc.VectorSubcoreMesh(core_axis_name="c", subcore_axis_name="s"),
        compiler_params=pltpu.CompilerParams(needs_layout_passes=False),
        out_type=jax.ShapeDtypeStruct((1, _N_EDGES), jnp.float32),
        scratch_types=[
            pltpu.VMEM((_Q_OFF + _N_NODES,), jnp.float32),
            pltpu.VMEM((2, _E_PER), jnp.int32),
            pltpu.VMEM((_E_PER,), jnp.float32),
            pltpu.SemaphoreType.DMA,
        ],
    )


def kernel(x, edge_label_index, W, b):
    table = _tc_prep(x, W, b)
    out = _sc_gather()(table, edge_label_index.astype(jnp.int32))
    return out.reshape(_N_EDGES, 1)


# speedup vs baseline: 64.3076x; 1.0056x over previous
import functools

import jax
import jax.numpy as jnp
from jax import lax
from jax.experimental import pallas as pl
from jax.experimental.pallas import tpu as pltpu
from jax.experimental.pallas import tpu_sc as plsc

_N_NODES = 10000
_D = 128
_N_EDGES = 320000

_NC = 2
_NS = 16
_NW = _NC * _NS
_LANES = 16
_E_PER = 10112
_LAST_LO = _N_EDGES - _E_PER
_Q_OFF = 10240


def _tc_body(x_ref, wt_ref, bias_ref, pq_ref):
    bias = jnp.concatenate(
        [bias_ref[...], jnp.zeros_like(bias_ref[...])], axis=0
    )
    pq = (
        lax.dot_general(
            wt_ref[...], x_ref[...],
            dimension_numbers=(((1,), (1,)), ((), ())),
            preferred_element_type=jnp.float32,
        )
        + bias
    )
    pq_ref[pl.ds(0, _N_NODES)] = pq[0:1, :].reshape(_N_NODES)
    pq_ref[pl.ds(_Q_OFF, _N_NODES)] = pq[1:2, :].reshape(_N_NODES)


def _tc_prep(x, W, b):
    wt = W.reshape(2, _D)
    bias = b.reshape(1, 1)
    return pl.pallas_call(
        _tc_body,
        out_shape=jax.ShapeDtypeStruct((_Q_OFF + _N_NODES,), jnp.float32),
    )(x, wt, bias)


def _sc_body(table_hbm, edges_hbm, out_hbm, tab_v, win_v, out_v, sem):
    wid = lax.axis_index("s") * _NC + lax.axis_index("c")
    base = jnp.minimum(wid * _E_PER, _LAST_LO)
    c0 = pltpu.async_copy(table_hbm, tab_v, sem)
    c1 = pltpu.async_copy(edges_hbm.at[:, pl.ds(base, _E_PER)], win_v, sem)
    c0.wait()
    c1.wait()

    def run(lo, hi):
        @plsc.parallel_loop(lo, hi, step=_LANES, unroll=8)
        def _(i):
            sl = pl.ds(i, _LANES)
            si = win_v[0, sl]
            di = win_v[1, sl]
            pv = plsc.load_gather(tab_v, [si])
            qv = plsc.load_gather(tab_v, [di + _Q_OFF])
            out_v[sl] = pv + qv

    cut = 5120
    run(0, cut)
    c3 = pltpu.async_copy(
        out_v.at[pl.ds(0, cut)], out_hbm.at[0, pl.ds(base, cut)], sem
    )
    run(cut, _E_PER)
    c4 = pltpu.async_copy(
        out_v.at[pl.ds(cut, _E_PER - cut)],
        out_hbm.at[0, pl.ds(base + cut, _E_PER - cut)],
        sem,
    )
    c3.wait()
    c4.wait()


@functools.cache
def _sc_gather():
    return pl.kernel(
        _sc_body,
        mesh=plsc.VectorSubcoreMesh(core_axis_name="c", subcore_axis_name="s"),
        compiler_params=pltpu.CompilerParams(needs_layout_passes=False),
        out_type=jax.ShapeDtypeStruct((1, _N_EDGES), jnp.float32),
        scratch_types=[
            pltpu.VMEM((_Q_OFF + _N_NODES,), jnp.float32),
            pltpu.VMEM((2, _E_PER), jnp.int32),
            pltpu.VMEM((_E_PER,), jnp.float32),
            pltpu.SemaphoreType.DMA,
        ],
    )


def kernel(x, edge_label_index, W, b):
    table = _tc_prep(x, W, b)
    out = _sc_gather()(table, edge_label_index.astype(jnp.int32))
    return out.reshape(_N_EDGES, 1)
